# baseline (speedup 1.0000x reference)
import functools

import jax
import jax.numpy as jnp
from jax import lax
from jax.experimental import pallas as pl
from jax.experimental.pallas import tpu as pltpu
from jax.experimental.pallas import tpu_sc as plsc

_E = 320000
_HID = 128
_NTYPE = 35
_NRID = 9
_NC = 2
_NS = 16
_NW = _NC * _NS
_PER_W = _E // _NW
_L = 16
_NB = 80
_GPB = _NB // _L
_CHUNKS = _PER_W // _NB
_BLK = 7 * _L
_BPT = _PER_W // _L


def _splat(vec, lane):
    return vec[jnp.full((_L,), lane, jnp.int32)]


def _sc_body(combo_h, pk_h, w_h, out_h,
             pk0, pk1, idx0, idx1, rows0, rows1, w_v,
             is0, is1, gs0, gs1, os0, os1):
    wid = lax.axis_index("s") * _NC + lax.axis_index("c")
    base = wid * _PER_W

    pltpu.sync_copy(w_h, w_v)
    pk = (pk0, pk1)
    idxv = (idx0, idx1)
    rows = (rows0, rows1)
    isem = (is0, is1)
    gsem = (gs0, gs1)
    osem = (os0, os1)

    def in_copy(c, d):
        bo = (wid * _BPT + c * _GPB) * _BLK
        n = _GPB * _BLK
        return pltpu.make_async_copy(
            pk_h.at[pl.ds(bo, n)], pk[d], isem[d])

    def gather_copy(d):
        return pltpu.make_async_copy(combo_h.at[idxv[d]], rows[d], gsem[d])

    def out_copy(c, d):
        cb = base + c * _NB
        return pltpu.make_async_copy(
            rows[d], out_h.at[pl.ds(cb, _NB)], osem[d])

    def compute_idx(d):
        for s in range(_NB // _L):
            et = pk[d][pl.ds(_BLK * s, _L)].astype(jnp.int32)
            ri = pk[d][pl.ds(_BLK * s + _L, _L)].astype(jnp.int32)
            idxv[d][pl.ds(_L * s, _L)] = et * _NRID + ri

    def fma_group(d, gg):
        av = [pk[d][pl.ds(_BLK * gg + _L * (2 + k), _L)] for k in range(5)]
        eb = gg * _L
        for h in range(2):
            wvh = [[w_v[pl.ds(128 * k + 64 * h + 16 * q, _L)]
                    for q in range(4)] for k in range(5)]
            for j in range(_L):
                cf = [_splat(av[k], j) for k in range(5)]
                for q in range(4):
                    col = 64 * h + 16 * q
                    r = rows[d][eb + j, pl.ds(col, _L)]
                    acc = (r + cf[0] * wvh[0][q] + cf[1] * wvh[1][q]
                           + cf[2] * wvh[2][q] + cf[3] * wvh[3][q]
                           + cf[4] * wvh[4][q])
                    rows[d][eb + j, pl.ds(col, _L)] = acc

    def fma(d):
        def grp(gg, carry):
            fma_group(d, gg)
            return carry
        lax.fori_loop(0, _GPB, grp, 0)

    def do_step(c, d, first=False, fire_gather=True, fire_in=True):
        dn = 1 - d
        if fire_gather:
            in_copy(c + 1, dn).wait()
            compute_idx(dn)
            if not first:
                out_copy(c - 1, dn).wait()
            gather_copy(dn).start()
        gather_copy(d).wait()
        fma(d)
        out_copy(c, d).start()
        if fire_in:
            in_copy(c + 2, d).start()

    in_copy(0, 0).start()
    in_copy(1, 1).start()
    in_copy(0, 0).wait()
    compute_idx(0)
    gather_copy(0).start()

    do_step(0, 0, first=True)

    def pair(i, carry):
        c = 2 * i
        do_step(c, 0)
        do_step(c + 1, 1)
        return carry

    do_step(1, 1)
    lax.fori_loop(1, (_CHUNKS - 3) // 2, pair, 0)
    do_step(_CHUNKS - 3, 0)
    do_step(_CHUNKS - 2, 1, fire_in=False)
    do_step(_CHUNKS - 1, 0, fire_gather=False, fire_in=False)

    out_copy(_CHUNKS - 2, 1).wait()
    out_copy(_CHUNKS - 1, 0).wait()


_sc_call = functools.partial(
    pl.kernel,
    out_type=jax.ShapeDtypeStruct((_E, _HID), jnp.float32),
    mesh=plsc.VectorSubcoreMesh(
        core_axis_name="c", subcore_axis_name="s",
        num_cores=_NC, num_subcores=_NS),
    scratch_types=[
        pltpu.VMEM((_GPB * _BLK,), jnp.float32),
        pltpu.VMEM((_GPB * _BLK,), jnp.float32),
        pltpu.VMEM((_NB,), jnp.int32),
        pltpu.VMEM((_NB,), jnp.int32),
        pltpu.VMEM((_NB, _HID), jnp.float32),
        pltpu.VMEM((_NB, _HID), jnp.float32),
        pltpu.VMEM((5 * _HID,), jnp.float32),
        pltpu.SemaphoreType.DMA,
        pltpu.SemaphoreType.DMA,
        pltpu.SemaphoreType.DMA,
        pltpu.SemaphoreType.DMA,
        pltpu.SemaphoreType.DMA,
        pltpu.SemaphoreType.DMA,
    ],
)(_sc_body)


@jax.jit
def kernel(etype, rid, att_rc, att_rp, W_type, W_rid, W_rc, b_rc, W_rp, b_rp):
    etype = etype.astype(jnp.int32)
    rid = rid.astype(jnp.int32)
    combo = ((W_type[:, None, :] + W_rid[None, :, :])
             .reshape(_NTYPE * _NRID, _HID) + b_rc + b_rp)
    wcat = jnp.concatenate([W_rc.T, W_rp.T], axis=0).reshape(-1)
    att_rp = att_rp.astype(jnp.float32)
    packed = (jnp.stack([etype.astype(jnp.float32), rid.astype(jnp.float32),
                         att_rc[:, 0], att_rc[:, 1],
                         att_rp[:, 0], att_rp[:, 1], att_rp[:, 2]], axis=0)
              .reshape(7, _E // _L, _L)
              .transpose(1, 0, 2)
              .reshape(-1))
    return _sc_call(combo, packed, wcat)


# speedup vs baseline: 1.2816x; 1.2816x over previous
import functools

import jax
import jax.numpy as jnp
from jax import lax
from jax.experimental import pallas as pl
from jax.experimental.pallas import tpu as pltpu
from jax.experimental.pallas import tpu_sc as plsc

_E = 320000
_HID = 128
_NTYPE = 35
_NRID = 9
_NC = 2
_NS = 16
_NW = _NC * _NS
_PER_W = _E // _NW
_NB = 80
_CHUNKS = _PER_W // _NB
_L = 16
_PKI = 2 * _NB
_PKF = 5 * _NB


def _splat(vec, lane):
    return vec[jnp.full((_L,), lane, jnp.int32)]


def _sc_body(combo_h, pki_h, pkf_h, w_h, out_h,
             pki0, pki1, pkf0, pkf1, idx0, idx1, rows0, rows1, w_v,
             is0, is1, gs0, gs1, os0, os1):
    wid = lax.axis_index("s") * _NC + lax.axis_index("c")
    base = wid * _PER_W

    pltpu.sync_copy(w_h, w_v)
    pki = (pki0, pki1)
    pkf = (pkf0, pkf1)
    idxv = (idx0, idx1)
    rows = (rows0, rows1)
    isem = (is0, is1)
    gsem = (gs0, gs1)
    osem = (os0, os1)

    def in_copies(c, d):
        gc = wid * _CHUNKS + c
        return (
            pltpu.make_async_copy(
                pki_h.at[pl.ds(gc * _PKI, _PKI)], pki[d], isem[d]),
            pltpu.make_async_copy(
                pkf_h.at[pl.ds(gc * _PKF, _PKF)], pkf[d], isem[d]),
        )

    def in_start(c, d):
        for cp in in_copies(c, d):
            cp.start()

    def in_wait(c, d):
        for cp in in_copies(c, d):
            cp.wait()

    def gather_copy(d):
        return pltpu.make_async_copy(combo_h.at[idxv[d]], rows[d], gsem[d])

    def out_copy(c, d):
        cb = base + c * _NB
        return pltpu.make_async_copy(
            rows[d], out_h.at[pl.ds(cb, _NB)], osem[d])

    def compute_idx(d):
        for s in range(_NB // _L):
            et = pki[d][pl.ds(_L * s, _L)]
            ri = pki[d][pl.ds(_NB + _L * s, _L)]
            idxv[d][pl.ds(_L * s, _L)] = et * _NRID + ri

    def fma(d):
        def grp(gg, carry):
            av = [pkf[d][pl.ds(k * _NB + _L * gg, _L)] for k in range(5)]
            eb = gg * _L
            for h in range(2):
                wvh = [[w_v[pl.ds(128 * k + 64 * h + 16 * q, _L)]
                        for q in range(4)] for k in range(5)]
                for j in range(_L):
                    cf = [_splat(av[k], j) for k in range(5)]
                    for q in range(4):
                        col = 64 * h + 16 * q
                        r = rows[d][eb + j, pl.ds(col, _L)]
                        acc = (r + cf[0] * wvh[0][q] + cf[1] * wvh[1][q]
                               + cf[2] * wvh[2][q] + cf[3] * wvh[3][q]
                               + cf[4] * wvh[4][q])
                        rows[d][eb + j, pl.ds(col, _L)] = acc
            return carry
        lax.fori_loop(0, _NB // _L, grp, 0)

    def do_step(c, d, first=False, fire_gather=True, fire_in=True):
        dn = 1 - d
        if fire_gather:
            in_wait(c + 1, dn)
            compute_idx(dn)
            if not first:
                out_copy(c - 1, dn).wait()
            gather_copy(dn).start()
        gather_copy(d).wait()
        fma(d)
        out_copy(c, d).start()
        if fire_in:
            in_start(c + 2, d)

    in_start(0, 0)
    in_start(1, 1)
    in_wait(0, 0)
    compute_idx(0)
    gather_copy(0).start()

    do_step(0, 0, first=True)

    def pair(i, carry):
        c = 2 * i
        do_step(c, 0)
        do_step(c + 1, 1)
        return carry

    do_step(1, 1)
    lax.fori_loop(1, (_CHUNKS - 3) // 2, pair, 0)
    do_step(_CHUNKS - 3, 0)
    do_step(_CHUNKS - 2, 1, fire_in=False)
    do_step(_CHUNKS - 1, 0, fire_gather=False, fire_in=False)

    out_copy(_CHUNKS - 2, 1).wait()
    out_copy(_CHUNKS - 1, 0).wait()


_sc_call = functools.partial(
    pl.kernel,
    out_type=jax.ShapeDtypeStruct((_E, _HID), jnp.float32),
    mesh=plsc.VectorSubcoreMesh(
        core_axis_name="c", subcore_axis_name="s",
        num_cores=_NC, num_subcores=_NS),
    scratch_types=[
        pltpu.VMEM((_PKI,), jnp.int32),
        pltpu.VMEM((_PKI,), jnp.int32),
        pltpu.VMEM((_PKF,), jnp.float32),
        pltpu.VMEM((_PKF,), jnp.float32),
        pltpu.VMEM((_NB,), jnp.int32),
        pltpu.VMEM((_NB,), jnp.int32),
        pltpu.VMEM((_NB, _HID), jnp.float32),
        pltpu.VMEM((_NB, _HID), jnp.float32),
        pltpu.VMEM((5 * _HID,), jnp.float32),
        pltpu.SemaphoreType.DMA,
        pltpu.SemaphoreType.DMA,
        pltpu.SemaphoreType.DMA,
        pltpu.SemaphoreType.DMA,
        pltpu.SemaphoreType.DMA,
        pltpu.SemaphoreType.DMA,
    ],
)(_sc_body)


@jax.jit
def kernel(etype, rid, att_rc, att_rp, W_type, W_rid, W_rc, b_rc, W_rp, b_rp):
    etype = etype.astype(jnp.int32)
    rid = rid.astype(jnp.int32)
    combo = ((W_type[:, None, :] + W_rid[None, :, :])
             .reshape(_NTYPE * _NRID, _HID) + b_rc + b_rp)
    wcat = jnp.concatenate([W_rc.T, W_rp.T], axis=0).reshape(-1)
    att_rp = att_rp.astype(jnp.float32)
    packed_i = (jnp.stack([etype, rid], axis=0)
                .reshape(2, _E // _NB, _NB)
                .transpose(1, 0, 2)
                .reshape(-1))
    packed_f = (jnp.stack([att_rc[:, 0], att_rc[:, 1],
                           att_rp[:, 0], att_rp[:, 1], att_rp[:, 2]], axis=0)
                .reshape(5, _E // _NB, _NB)
                .transpose(1, 0, 2)
                .reshape(-1))
    return _sc_call(combo, packed_i, packed_f, wcat)


# speedup vs baseline: 1.5270x; 1.1915x over previous
import functools

import jax
import jax.numpy as jnp
from jax import lax
from jax.experimental import pallas as pl
from jax.experimental.pallas import tpu as pltpu
from jax.experimental.pallas import tpu_sc as plsc

_E = 320000
_HID = 128
_NTYPE = 35
_NRID = 9
_NC = 2
_NS = 16
_NW = _NC * _NS
_PER_W = _E // _NW
_NB = 80
_CHUNKS = _PER_W // _NB
_L = 16


def _splat(vec, lane):
    return vec[jnp.full((_L,), lane, jnp.int32)]


def _sc_body(combo_h, et_h, ri_h, a0_h, a1_h, a2_h, a3_h, a4_h, w_h, out_h,
             pki0, pki1, pkf0, pkf1, idx0, idx1, rows0, rows1, w_v,
             is0, is1, gs0, gs1, os0, os1):
    wid = lax.axis_index("s") * _NC + lax.axis_index("c")
    base = wid * _PER_W

    pltpu.sync_copy(w_h, w_v)
    pki = (pki0, pki1)
    pkf = (pkf0, pkf1)
    idxv = (idx0, idx1)
    rows = (rows0, rows1)
    isem = (is0, is1)
    gsem = (gs0, gs1)
    osem = (os0, os1)

    def in_copies(c, d):
        cb = base + c * _NB
        ins = [pltpu.make_async_copy(
                   h.at[pl.ds(cb, _NB)], pki[d].at[pl.ds(k * _NB, _NB)],
                   isem[d])
               for k, h in enumerate((et_h, ri_h))]
        ins += [pltpu.make_async_copy(
                    h.at[pl.ds(cb, _NB)], pkf[d].at[pl.ds(k * _NB, _NB)],
                    isem[d])
                for k, h in enumerate((a0_h, a1_h, a2_h, a3_h, a4_h))]
        return ins

    def in_start(c, d):
        for cp in in_copies(c, d):
            cp.start()

    def in_wait(c, d):
        for cp in in_copies(c, d):
            cp.wait()

    def gather_copy(d):
        return pltpu.make_async_copy(combo_h.at[idxv[d]], rows[d], gsem[d])

    def out_copy(c, d):
        cb = base + c * _NB
        return pltpu.make_async_copy(
            rows[d], out_h.at[pl.ds(cb, _NB)], osem[d])

    def compute_idx(d):
        for s in range(_NB // _L):
            et = pki[d][pl.ds(_L * s, _L)]
            ri = pki[d][pl.ds(_NB + _L * s, _L)]
            idxv[d][pl.ds(_L * s, _L)] = et * _NRID + ri

    def fma(d):
        def grp(gg, carry):
            av = [pkf[d][pl.ds(k * _NB + _L * gg, _L)] for k in range(5)]
            eb = gg * _L
            for h in range(2):
                wvh = [[w_v[pl.ds(128 * k + 64 * h + 16 * q, _L)]
                        for q in range(4)] for k in range(5)]
                for j in range(_L):
                    cf = [_splat(av[k], j) for k in range(5)]
                    for q in range(4):
                        col = 64 * h + 16 * q
                        r = rows[d][eb + j, pl.ds(col, _L)]
                        acc = (r + cf[0] * wvh[0][q] + cf[1] * wvh[1][q]
                               + cf[2] * wvh[2][q] + cf[3] * wvh[3][q]
                               + cf[4] * wvh[4][q])
                        rows[d][eb + j, pl.ds(col, _L)] = acc
            return carry
        lax.fori_loop(0, _NB // _L, grp, 0)

    def do_step(c, d, first=False, fire_gather=True, fire_in=True):
        dn = 1 - d
        if fire_gather:
            in_wait(c + 1, dn)
            compute_idx(dn)
            if not first:
                out_copy(c - 1, dn).wait()
            gather_copy(dn).start()
        gather_copy(d).wait()
        fma(d)
        out_copy(c, d).start()
        if fire_in:
            in_start(c + 2, d)

    in_start(0, 0)
    in_start(1, 1)
    in_wait(0, 0)
    compute_idx(0)
    gather_copy(0).start()

    do_step(0, 0, first=True)

    def pair(i, carry):
        c = 2 * i
        do_step(c, 0)
        do_step(c + 1, 1)
        return carry

    do_step(1, 1)
    lax.fori_loop(1, (_CHUNKS - 3) // 2, pair, 0)
    do_step(_CHUNKS - 3, 0)
    do_step(_CHUNKS - 2, 1, fire_in=False)
    do_step(_CHUNKS - 1, 0, fire_gather=False, fire_in=False)

    out_copy(_CHUNKS - 2, 1).wait()
    out_copy(_CHUNKS - 1, 0).wait()


_sc_call = functools.partial(
    pl.kernel,
    out_type=jax.ShapeDtypeStruct((_E, _HID), jnp.float32),
    mesh=plsc.VectorSubcoreMesh(
        core_axis_name="c", subcore_axis_name="s",
        num_cores=_NC, num_subcores=_NS),
    scratch_types=[
        pltpu.VMEM((2 * _NB,), jnp.int32),
        pltpu.VMEM((2 * _NB,), jnp.int32),
        pltpu.VMEM((5 * _NB,), jnp.float32),
        pltpu.VMEM((5 * _NB,), jnp.float32),
        pltpu.VMEM((_NB,), jnp.int32),
        pltpu.VMEM((_NB,), jnp.int32),
        pltpu.VMEM((_NB, _HID), jnp.float32),
        pltpu.VMEM((_NB, _HID), jnp.float32),
        pltpu.VMEM((5 * _HID,), jnp.float32),
        pltpu.SemaphoreType.DMA,
        pltpu.SemaphoreType.DMA,
        pltpu.SemaphoreType.DMA,
        pltpu.SemaphoreType.DMA,
        pltpu.SemaphoreType.DMA,
        pltpu.SemaphoreType.DMA,
    ],
)(_sc_body)


@jax.jit
def kernel(etype, rid, att_rc, att_rp, W_type, W_rid, W_rc, b_rc, W_rp, b_rp):
    etype = etype.astype(jnp.int32)
    rid = rid.astype(jnp.int32)
    combo = ((W_type[:, None, :] + W_rid[None, :, :])
             .reshape(_NTYPE * _NRID, _HID) + b_rc + b_rp)
    wcat = jnp.concatenate([W_rc.T, W_rp.T], axis=0).reshape(-1)
    att_rp = att_rp.astype(jnp.float32)
    return _sc_call(combo, etype, rid,
                    att_rc[:, 0], att_rc[:, 1],
                    att_rp[:, 0], att_rp[:, 1], att_rp[:, 2], wcat)


# speedup vs baseline: 1.5272x; 1.0001x over previous
import functools

import jax
import jax.numpy as jnp
from jax import lax
from jax.experimental import pallas as pl
from jax.experimental.pallas import tpu as pltpu
from jax.experimental.pallas import tpu_sc as plsc

_E = 320000
_HID = 128
_NTYPE = 35
_NRID = 9
_NC = 2
_NS = 16
_NW = _NC * _NS
_PER_W = _E // _NW
_NB = 80
_CHUNKS = _PER_W // _NB
_L = 16


def _splat(vec, lane):
    return vec[jnp.full((_L,), lane, jnp.int32)]


def _sc_body(combo_h, et_h, ri_h, a0_h, a1_h, a2_h, a3_h, a4_h, w_h, out_h,
             pki0, pki1, pkf0, pkf1, idx0, idx1, rows0, rows1, w_v,
             is0, is1, gs0, gs1, os0, os1):
    wid = lax.axis_index("s") * _NC + lax.axis_index("c")
    base = wid * _PER_W

    pltpu.sync_copy(w_h, w_v)
    pki = (pki0, pki1)
    pkf = (pkf0, pkf1)
    idxv = (idx0, idx1)
    rows = (rows0, rows1)
    isem = (is0, is1)
    gsem = (gs0, gs1)
    osem = (os0, os1)

    def in_copies(c, d):
        cb = base + c * _NB
        ins = [pltpu.make_async_copy(
                   h.at[pl.ds(cb, _NB)], pki[d].at[pl.ds(k * _NB, _NB)],
                   isem[d])
               for k, h in enumerate((et_h, ri_h))]
        ins += [pltpu.make_async_copy(
                    h.at[pl.ds(cb, _NB)], pkf[d].at[pl.ds(k * _NB, _NB)],
                    isem[d])
                for k, h in enumerate((a0_h, a1_h, a2_h, a3_h, a4_h))]
        return ins

    def in_start(c, d):
        for cp in in_copies(c, d):
            cp.start()

    def in_wait(c, d):
        pltpu.make_async_copy(
            et_h.at[pl.ds(0, 2 * _NB)], pki[d], isem[d]).wait()
        pltpu.make_async_copy(
            a0_h.at[pl.ds(0, 5 * _NB)], pkf[d], isem[d]).wait()

    def gather_copy(d):
        return pltpu.make_async_copy(combo_h.at[idxv[d]], rows[d], gsem[d])

    def out_copy(c, d):
        cb = base + c * _NB
        return pltpu.make_async_copy(
            rows[d], out_h.at[pl.ds(cb, _NB)], osem[d])

    def compute_idx(d):
        for s in range(_NB // _L):
            et = pki[d][pl.ds(_L * s, _L)]
            ri = pki[d][pl.ds(_NB + _L * s, _L)]
            idxv[d][pl.ds(_L * s, _L)] = et * _NRID + ri

    def fma(d):
        def grp(gg, carry):
            av = [pkf[d][pl.ds(k * _NB + _L * gg, _L)] for k in range(5)]
            eb = gg * _L
            for h in range(2):
                wvh = [[w_v[pl.ds(128 * k + 64 * h + 16 * q, _L)]
                        for q in range(4)] for k in range(5)]
                for j in range(_L):
                    cf = [_splat(av[k], j) for k in range(5)]
                    for q in range(4):
                        col = 64 * h + 16 * q
                        r = rows[d][eb + j, pl.ds(col, _L)]
                        acc = (r + cf[0] * wvh[0][q] + cf[1] * wvh[1][q]
                               + cf[2] * wvh[2][q] + cf[3] * wvh[3][q]
                               + cf[4] * wvh[4][q])
                        rows[d][eb + j, pl.ds(col, _L)] = acc
            return carry
        lax.fori_loop(0, _NB // _L, grp, 0)

    def do_step(c, d, first=False, fire_gather=True, fire_in=True):
        dn = 1 - d
        if fire_gather:
            in_wait(c + 1, dn)
            compute_idx(dn)
            if not first:
                out_copy(c - 1, dn).wait()
            gather_copy(dn).start()
        gather_copy(d).wait()
        fma(d)
        out_copy(c, d).start()
        if fire_in:
            in_start(c + 2, d)

    in_start(0, 0)
    in_start(1, 1)
    in_wait(0, 0)
    compute_idx(0)
    gather_copy(0).start()

    do_step(0, 0, first=True)

    def pair(i, carry):
        c = 2 * i
        do_step(c, 0)
        do_step(c + 1, 1)
        return carry

    do_step(1, 1)
    lax.fori_loop(1, (_CHUNKS - 3) // 2, pair, 0)
    do_step(_CHUNKS - 3, 0)
    do_step(_CHUNKS - 2, 1, fire_in=False)
    do_step(_CHUNKS - 1, 0, fire_gather=False, fire_in=False)

    out_copy(_CHUNKS - 2, 1).wait()
    out_copy(_CHUNKS - 1, 0).wait()


_sc_call = functools.partial(
    pl.kernel,
    out_type=jax.ShapeDtypeStruct((_E, _HID), jnp.float32),
    mesh=plsc.VectorSubcoreMesh(
        core_axis_name="c", subcore_axis_name="s",
        num_cores=_NC, num_subcores=_NS),
    scratch_types=[
        pltpu.VMEM((2 * _NB,), jnp.int32),
        pltpu.VMEM((2 * _NB,), jnp.int32),
        pltpu.VMEM((5 * _NB,), jnp.float32),
        pltpu.VMEM((5 * _NB,), jnp.float32),
        pltpu.VMEM((_NB,), jnp.int32),
        pltpu.VMEM((_NB,), jnp.int32),
        pltpu.VMEM((_NB, _HID), jnp.float32),
        pltpu.VMEM((_NB, _HID), jnp.float32),
        pltpu.VMEM((5 * _HID,), jnp.float32),
        pltpu.SemaphoreType.DMA,
        pltpu.SemaphoreType.DMA,
        pltpu.SemaphoreType.DMA,
        pltpu.SemaphoreType.DMA,
        pltpu.SemaphoreType.DMA,
        pltpu.SemaphoreType.DMA,
    ],
)(_sc_body)


@jax.jit
def kernel(etype, rid, att_rc, att_rp, W_type, W_rid, W_rc, b_rc, W_rp, b_rp):
    etype = etype.astype(jnp.int32)
    rid = rid.astype(jnp.int32)
    combo = ((W_type[:, None, :] + W_rid[None, :, :])
             .reshape(_NTYPE * _NRID, _HID) + b_rc + b_rp)
    wcat = jnp.concatenate([W_rc.T, W_rp.T], axis=0).reshape(-1)
    att_rp = att_rp.astype(jnp.float32)
    return _sc_call(combo, etype, rid,
                    att_rc[:, 0], att_rc[:, 1],
                    att_rp[:, 0], att_rp[:, 1], att_rp[:, 2], wcat)


# speedup vs baseline: 1.5548x; 1.0181x over previous
import functools

import jax
import jax.numpy as jnp
from jax import lax
from jax.experimental import pallas as pl
from jax.experimental.pallas import tpu as pltpu
from jax.experimental.pallas import tpu_sc as plsc

_E = 320000
_HID = 128
_NTYPE = 35
_NRID = 9
_NC = 2
_NS = 16
_NW = _NC * _NS
_PER_W = _E // _NW
_NB = 128
_FULL = _PER_W // _NB
_TAIL = _PER_W - _FULL * _NB
_L = 16


def _splat(vec, lane):
    return vec[jnp.full((_L,), lane, jnp.int32)]


def _sc_body(combo_h, et_h, ri_h, a0_h, a1_h, a2_h, a3_h, a4_h, w_h, out_h,
             pki0, pki1, pkf0, pkf1, idx0, idx1, rows0, rows1, w_v,
             is0, is1, gs0, gs1, os0, os1):
    wid = lax.axis_index("s") * _NC + lax.axis_index("c")
    base = wid * _PER_W

    pltpu.sync_copy(w_h, w_v)
    pki = (pki0, pki1)
    pkf = (pkf0, pkf1)
    idxv = (idx0, idx1)
    rows = (rows0, rows1)
    isem = (is0, is1)
    gsem = (gs0, gs1)
    osem = (os0, os1)

    def in_start(c, d, nb=_NB):
        cb = base + c * _NB
        for k, h in enumerate((et_h, ri_h)):
            pltpu.make_async_copy(
                h.at[pl.ds(cb, nb)], pki[d].at[pl.ds(k * _NB, nb)],
                isem[d]).start()
        for k, h in enumerate((a0_h, a1_h, a2_h, a3_h, a4_h)):
            pltpu.make_async_copy(
                h.at[pl.ds(cb, nb)], pkf[d].at[pl.ds(k * _NB, nb)],
                isem[d]).start()

    def in_wait(c, d, nb=_NB):
        pltpu.make_async_copy(
            et_h.at[pl.ds(0, 2 * nb)], pki[d].at[pl.ds(0, 2 * nb)],
            isem[d]).wait()
        pltpu.make_async_copy(
            a0_h.at[pl.ds(0, 5 * nb)], pkf[d].at[pl.ds(0, 5 * nb)],
            isem[d]).wait()

    def gather_copy(d, nb=_NB):
        if nb == _NB:
            return pltpu.make_async_copy(
                combo_h.at[idxv[d]], rows[d], gsem[d])
        return pltpu.make_async_copy(
            combo_h.at[idxv[d].at[pl.ds(0, nb)]],
            rows[d].at[pl.ds(0, nb)], gsem[d])

    def out_copy(c, d, nb=_NB):
        cb = base + c * _NB
        return pltpu.make_async_copy(
            rows[d].at[pl.ds(0, nb)], out_h.at[pl.ds(cb, nb)], osem[d])

    def compute_idx(d, nb=_NB):
        for s in range(nb // _L):
            et = pki[d][pl.ds(_L * s, _L)]
            ri = pki[d][pl.ds(_NB + _L * s, _L)]
            idxv[d][pl.ds(_L * s, _L)] = et * _NRID + ri

    def fma(d, nb=_NB):
        def grp(gg, carry):
            av = [pkf[d][pl.ds(k * _NB + _L * gg, _L)] for k in range(5)]
            eb = gg * _L
            for h in range(2):
                wvh = [[w_v[pl.ds(128 * k + 64 * h + 16 * q, _L)]
                        for q in range(4)] for k in range(5)]
                for j in range(_L):
                    cf = [_splat(av[k], j) for k in range(5)]
                    for q in range(4):
                        col = 64 * h + 16 * q
                        r = rows[d][eb + j, pl.ds(col, _L)]
                        acc = (r + cf[0] * wvh[0][q] + cf[1] * wvh[1][q]
                               + cf[2] * wvh[2][q] + cf[3] * wvh[3][q]
                               + cf[4] * wvh[4][q])
                        rows[d][eb + j, pl.ds(col, _L)] = acc
            return carry
        lax.fori_loop(0, nb // _L, grp, 0)

    def do_step(c, d, first=False, fire_in=True, in_nb=_NB, next_nb=_NB):
        dn = 1 - d
        in_wait(c + 1, dn, next_nb)
        compute_idx(dn, next_nb)
        if not first:
            out_copy(c - 1, dn).wait()
        gather_copy(dn, next_nb).start()
        gather_copy(d).wait()
        fma(d)
        out_copy(c, d).start()
        if fire_in:
            in_start(c + 2, d, in_nb)

    in_start(0, 0)
    in_start(1, 1)
    in_wait(0, 0)
    compute_idx(0)
    gather_copy(0).start()

    do_step(0, 0, first=True)

    def pair(i, carry):
        c = 2 * i
        do_step(c, 0)
        do_step(c + 1, 1)
        return carry

    do_step(1, 1)
    lax.fori_loop(1, (_FULL - 2) // 2, pair, 0)
    do_step(_FULL - 2, 0, in_nb=_TAIL)
    do_step(_FULL - 1, 1, fire_in=False, next_nb=_TAIL)

    gather_copy(0, _TAIL).wait()
    fma(0, _TAIL)
    out_copy(_FULL, 0, _TAIL).start()

    out_copy(_FULL - 1, 1).wait()
    out_copy(_FULL, 0, _TAIL).wait()


_sc_call = functools.partial(
    pl.kernel,
    out_type=jax.ShapeDtypeStruct((_E, _HID), jnp.float32),
    mesh=plsc.VectorSubcoreMesh(
        core_axis_name="c", subcore_axis_name="s",
        num_cores=_NC, num_subcores=_NS),
    scratch_types=[
        pltpu.VMEM((2 * _NB,), jnp.int32),
        pltpu.VMEM((2 * _NB,), jnp.int32),
        pltpu.VMEM((5 * _NB,), jnp.float32),
        pltpu.VMEM((5 * _NB,), jnp.float32),
        pltpu.VMEM((_NB,), jnp.int32),
        pltpu.VMEM((_NB,), jnp.int32),
        pltpu.VMEM((_NB, _HID), jnp.float32),
        pltpu.VMEM((_NB, _HID), jnp.float32),
        pltpu.VMEM((5 * _HID,), jnp.float32),
        pltpu.SemaphoreType.DMA,
        pltpu.SemaphoreType.DMA,
        pltpu.SemaphoreType.DMA,
        pltpu.SemaphoreType.DMA,
        pltpu.SemaphoreType.DMA,
        pltpu.SemaphoreType.DMA,
    ],
)(_sc_body)


@jax.jit
def kernel(etype, rid, att_rc, att_rp, W_type, W_rid, W_rc, b_rc, W_rp, b_rp):
    etype = etype.astype(jnp.int32)
    rid = rid.astype(jnp.int32)
    combo = ((W_type[:, None, :] + W_rid[None, :, :])
             .reshape(_NTYPE * _NRID, _HID) + b_rc + b_rp)
    wcat = jnp.concatenate([W_rc.T, W_rp.T], axis=0).reshape(-1)
    att_rp = att_rp.astype(jnp.float32)
    return _sc_call(combo, etype, rid,
                    att_rc[:, 0], att_rc[:, 1],
                    att_rp[:, 0], att_rp[:, 1], att_rp[:, 2], wcat)
